# baseline (speedup 1.0000x reference)
import jax
import jax.numpy as jnp
from jax import lax
from jax.experimental import pallas as pl
from jax.experimental.pallas import tpu as pltpu
from jax.experimental.pallas import tpu_sc as plsc

_BATCH = 8
_NPTS = 1024
_DF = 3
_NN = 16
_R = 256
_NB = _NPTS // _R
_PAD = 16
_BIG = 1e30


def _topk_body(x3_ref, xTq_ref, idx_ref, x3p_ref):
    b = pl.program_id(0)
    X = x3_ref[0]
    Q = xTq_ref[0]
    d0 = X[:, 0:1] - Q[0:1, :]
    d1 = X[:, 1:2] - Q[1:2, :]
    d2 = X[:, 2:3] - Q[2:3, :]
    D = d0 * d0 + d1 * d1 + d2 * d2
    D = jnp.where(D == 0.0, _BIG, D)

    iota = lax.broadcasted_iota(jnp.int32, (_NPTS, _R), 0)
    rows = []
    for s in range(_NN):
        m = jnp.min(D, axis=0, keepdims=True)
        jsel = jnp.where(D == m, iota, jnp.int32(2 ** 30))
        jmin = jnp.min(jsel, axis=0, keepdims=True)
        rows.append(jmin)
        if s < _NN - 1:
            D = jnp.where(jsel == jmin, _BIG, D)
    idx_ref[0, 0] = jnp.concatenate(rows, axis=0) + b * _NPTS

    @pl.when(pl.program_id(1) == 0)
    def _pad_rows():
        x3p_ref[...] = jnp.concatenate(
            [X, jnp.zeros((_NPTS, _PAD - _DF), jnp.float32)], axis=1)


def _run_topk_tc(x3, xT):
    return pl.pallas_call(
        _topk_body,
        grid=(_BATCH, _NB),
        in_specs=[
            pl.BlockSpec((1, _NPTS, _DF), lambda b, r: (b, 0, 0)),
            pl.BlockSpec((1, _DF, _R), lambda b, r: (b, 0, r)),
        ],
        out_specs=[
            pl.BlockSpec((1, 1, _NN, _R), lambda b, r: (b, r, 0, 0)),
            pl.BlockSpec((_NPTS, _PAD), lambda b, r: (b, 0)),
        ],
        out_shape=[
            jax.ShapeDtypeStruct((_BATCH, _NB, _NN, _R), jnp.int32),
            jax.ShapeDtypeStruct((_BATCH * _NPTS, _PAD), jnp.float32),
        ],
        compiler_params=pltpu.CompilerParams(
            dimension_semantics=("arbitrary", "arbitrary"),
        ),
    )(x3, xT)


def _gather_sc(x3p_hbm, idx_hbm, out_hbm, ibuf_v, gbuf_v, semi, semg):
    wid = lax.axis_index("s") * 2 + lax.axis_index("c")
    b = wid // _NB
    rb = wid % _NB

    nseg = (_R * _NN) // 128
    hs = []
    for r in range(nseg):
        hs.append(pltpu.async_copy(
            idx_hbm.at[b, rb, r // 2, pl.ds((r % 2) * 128, 128)],
            ibuf_v.at[r], semi))
    for h in hs:
        h.wait()
    hs = []
    for r in range(nseg):
        hs.append(pltpu.async_copy(
            x3p_hbm.at[ibuf_v.at[r]],
            gbuf_v.at[pl.ds(r * 128, 128)], semg))
    for h in hs:
        h.wait()
    pltpu.sync_copy(gbuf_v, out_hbm.at[b, pl.ds(rb * _R * _NN, _R * _NN)])


def _run_gather_sc(x3p, idxg):
    mesh = plsc.VectorSubcoreMesh(core_axis_name="c", subcore_axis_name="s")
    return pl.kernel(
        _gather_sc,
        mesh=mesh,
        out_type=jax.ShapeDtypeStruct((_BATCH, _NPTS * _NN, _PAD), jnp.float32),
        scratch_types=[
            pltpu.VMEM((32, 128), jnp.int32),
            pltpu.VMEM((_R * _NN, _PAD), jnp.float32),
            pltpu.SemaphoreType.DMA,
            pltpu.SemaphoreType.DMA,
        ],
        compiler_params=pltpu.CompilerParams(use_tc_tiling_on_sc=False),
    )(x3p, idxg)


def _mlp_body(nb_ref, z_ref, A_ref, B_ref, C_ref, E_ref, b4_ref,
              Wcat_ref, bcat_ref, Wzc_ref, bzc_ref,
              xout_ref, zout_ref):
    rb = pl.program_id(1)
    nb4 = nb_ref[0]

    def stack(cols):
        return jnp.concatenate(cols, axis=0)

    relu = lambda a: jnp.maximum(a, 0.0)
    third = jnp.float32(1.0 / 3.0)

    A = A_ref[...]
    B = B_ref[...]
    C = C_ref[...]
    E = E_ref[...]
    b4 = b4_ref[...]
    F = jnp.zeros((15 * _R, 128), jnp.float32)
    for f in range(_DF):
        cf = [nb4[s, :, f:f + 1] for s in range(_NN)]
        t1 = stack([cf[0]] * 15)
        t2 = stack(cf[1:16])
        u2 = stack([cf[1]] * 14 + [cf[2]])
        u3 = stack(cf[2:16] + [cf[3]])
        F = F + relu(t1 * A + t2 * B + u2 * C + u3 * E + b4)

    zsc = z_ref[0, 0, 0]
    zcat = zsc * Wzc_ref[...] + bzc_ref[...]

    XV = jnp.dot(F * third, Wcat_ref[...], preferred_element_type=jnp.float32)
    XV = relu(XV + bcat_ref[...] + zcat)

    xm = XV[:, 0:32]
    xs = xm[0:_R]
    for k in range(1, 15):
        xs = xs + xm[k * _R:(k + 1) * _R]
    xout_ref[0] = xs * jnp.float32(1.0 / 15.0)

    vm = XV[:, 32:96]
    part = jnp.sum(vm, axis=0, keepdims=True)
    acc = jnp.where(rb == 0, part, zout_ref[0] + part)
    zout_ref[0] = jnp.where(rb == _NB - 1,
                            acc * jnp.float32(1.0 / (15.0 * _NPTS)), acc)


def kernel(x, z, Wm2, bm2, Wm3, bm3, Wv2, bv2, Wv3, bv3,
           Wmx, bmx, Wvx, bvx, Wmz, bmz, Wvz, bvz):
    bs = x.shape[0]
    x3 = x.reshape(bs, _NPTS, _DF)
    xT = jnp.zeros((bs, _DF, _NPTS), jnp.float32)

    idxg, x3p = _run_topk_tc(x3, xT)
    neigh = _run_gather_sc(x3p, idxg)
    neigh4 = neigh.reshape(bs, _NB * _NN, _R, _PAD)

    row = lambda a: a.reshape(1, -1)
    z32 = jnp.zeros((32,), jnp.float32)
    cat = lambda parts: jnp.concatenate(parts).reshape(1, 128)
    A_row = cat([Wm2[:, 0], Wm3[:, 0], Wv2[:, 0], Wv3[:, 0]])
    B_row = cat([Wm2[:, 1], z32, Wv2[:, 1], z32])
    C_row = cat([z32, Wm3[:, 1], z32, Wv3[:, 1]])
    E_row = cat([z32, Wm3[:, 2], z32, Wv3[:, 2]])
    b4_row = cat([bm2, bm3, bv2, bv3])
    Wcat = jnp.zeros((128, 96), jnp.float32)
    Wcat = Wcat.at[0:64, 0:32].set(Wmx.T).at[64:128, 32:96].set(Wvx.T)
    bcat = jnp.concatenate([bmx, bvx]).reshape(1, 96)
    Wzc = jnp.concatenate([Wmz.T, Wvz.T], axis=1)
    bzc = jnp.concatenate([bmz, bvz]).reshape(1, 96)
    full = lambda shp: pl.BlockSpec(shp, lambda b, r: (0,) * len(shp))

    xout, zout = pl.pallas_call(
        _mlp_body,
        grid=(bs, _NB),
        in_specs=[
            pl.BlockSpec((1, _NN, _R, _PAD), lambda b, r: (b, r, 0, 0)),
            pl.BlockSpec((1, 1, 1), lambda b, r: (b, 0, 0)),
            full((1, 128)), full((1, 128)), full((1, 128)), full((1, 128)),
            full((1, 128)),
            full((128, 96)), full((1, 96)),
            full((1, 96)), full((1, 96)),
        ],
        out_specs=[
            pl.BlockSpec((1, _R, 32), lambda b, r: (b, r, 0)),
            pl.BlockSpec((1, 1, 64), lambda b, r: (b, 0, 0)),
        ],
        out_shape=[
            jax.ShapeDtypeStruct((bs, _NPTS, 32), jnp.float32),
            jax.ShapeDtypeStruct((bs, 1, 64), jnp.float32),
        ],
        compiler_params=pltpu.CompilerParams(
            dimension_semantics=("arbitrary", "arbitrary"),
        ),
    )(neigh4, z.reshape(bs, 1, 1),
      A_row, B_row, C_row, E_row, b4_row, Wcat, bcat, Wzc, bzc)

    return xout.reshape(bs, _NPTS * 32), zout.reshape(bs, 64)


# speedup vs baseline: 1.1329x; 1.1329x over previous
import jax
import jax.numpy as jnp
from jax import lax
from jax.experimental import pallas as pl
from jax.experimental.pallas import tpu as pltpu
from jax.experimental.pallas import tpu_sc as plsc

_BATCH = 8
_NPTS = 1024
_DF = 3
_NN = 16
_R = 256
_NB = _NPTS // _R
_PAD = 16
_BIG = 1e30


def _topk_body(x3_ref, xT_ref, idx_ref, x3p_ref):
    b = pl.program_id(0)
    q = x3_ref[0]
    X0 = xT_ref[0, 0:1, :]
    X1 = xT_ref[0, 1:2, :]
    X2 = xT_ref[0, 2:3, :]
    d0 = q[:, 0:1] - X0
    d1 = q[:, 1:2] - X1
    d2 = q[:, 2:3] - X2
    D = d0 * d0 + d1 * d1 + d2 * d2
    D = jnp.where(D == 0.0, _BIG, D)

    iota = lax.broadcasted_iota(jnp.int32, (_R, _NPTS), 1).astype(jnp.float32)
    cols = []
    for s in range(_NN):
        m = jnp.min(D, axis=1, keepdims=True)
        jsel = jnp.where(D == m, iota, jnp.float32(2.0 ** 30))
        jmin = jnp.min(jsel, axis=1, keepdims=True)
        cols.append(jmin)
        if s < _NN - 1:
            D = jnp.where(jsel == jmin, _BIG, D)
    idx = jnp.concatenate(cols, axis=1).astype(jnp.int32)
    idx_ref[0] = idx + b * _NPTS

    x3p_ref[...] = jnp.concatenate(
        [q, jnp.zeros((_R, _PAD - _DF), jnp.float32)], axis=1)


def _run_topk_tc(x3, xT):
    return pl.pallas_call(
        _topk_body,
        grid=(_BATCH, _NB),
        in_specs=[
            pl.BlockSpec((1, _R, _DF), lambda b, r: (b, r, 0)),
            pl.BlockSpec((1, _DF, _NPTS), lambda b, r: (b, 0, 0)),
        ],
        out_specs=[
            pl.BlockSpec((1, _R, _NN), lambda b, r: (b, r, 0)),
            pl.BlockSpec((_R, _PAD), lambda b, r: (b * _NB + r, 0)),
        ],
        out_shape=[
            jax.ShapeDtypeStruct((_BATCH, _NPTS, _NN), jnp.int32),
            jax.ShapeDtypeStruct((_BATCH * _NPTS, _PAD), jnp.float32),
        ],
        compiler_params=pltpu.CompilerParams(
            dimension_semantics=("arbitrary", "arbitrary"),
        ),
    )(x3, xT)


def _gather_sc(x3p_hbm, idx_hbm, out_hbm, ibuf_v, gbuf_v, semi, semg):
    wid = lax.axis_index("s") * 2 + lax.axis_index("c")
    b = wid // _NB
    rb = wid % _NB

    nseg = (_R * _NN) // 128
    base = (b * _NPTS + rb * _R) * _NN
    hs = []
    for r in range(nseg):
        hs.append(pltpu.async_copy(
            idx_hbm.at[pl.ds(base + r * 128, 128)],
            ibuf_v.at[r], semi))
    for h in hs:
        h.wait()
    hs = []
    for r in range(nseg):
        hs.append(pltpu.async_copy(
            x3p_hbm.at[ibuf_v.at[r]],
            gbuf_v.at[pl.ds(r * 128, 128)], semg))
    for h in hs:
        h.wait()
    pltpu.sync_copy(gbuf_v, out_hbm.at[b, pl.ds(rb * _R * _NN, _R * _NN)])


def _run_gather_sc(x3p, idxg):
    mesh = plsc.VectorSubcoreMesh(core_axis_name="c", subcore_axis_name="s")
    return pl.kernel(
        _gather_sc,
        mesh=mesh,
        out_type=jax.ShapeDtypeStruct((_BATCH, _NPTS * _NN, _PAD), jnp.float32),
        scratch_types=[
            pltpu.VMEM((32, 128), jnp.int32),
            pltpu.VMEM((_R * _NN, _PAD), jnp.float32),
            pltpu.SemaphoreType.DMA,
            pltpu.SemaphoreType.DMA,
        ],
        compiler_params=pltpu.CompilerParams(use_tc_tiling_on_sc=False),
    )(x3p, idxg)


def _mlp_body(nb_ref, z_ref, A_ref, B_ref, C_ref, E_ref, b4_ref,
              Wcat_ref, bcat_ref, Wzc_ref, bzc_ref,
              xout_ref, zout_ref):
    rb = pl.program_id(1)
    nb4 = nb_ref[0]

    def stack(cols):
        return jnp.concatenate(cols, axis=0)

    relu = lambda a: jnp.maximum(a, 0.0)
    third = jnp.float32(1.0 / 3.0)

    A = A_ref[...]
    B = B_ref[...]
    C = C_ref[...]
    E = E_ref[...]
    b4 = b4_ref[...]
    F = jnp.zeros((15 * _R, 128), jnp.float32)
    for f in range(_DF):
        cf = [nb4[:, s, f:f + 1] for s in range(_NN)]
        t1 = stack([cf[0]] * 15)
        t2 = stack(cf[1:16])
        u2 = stack([cf[1]] * 14 + [cf[2]])
        u3 = stack(cf[2:16] + [cf[3]])
        F = F + relu(t1 * A + t2 * B + u2 * C + u3 * E + b4)

    zsc = z_ref[0, 0, 0]
    zcat = zsc * Wzc_ref[...] + bzc_ref[...]

    XV = jnp.dot(F * third, Wcat_ref[...], preferred_element_type=jnp.float32)
    XV = relu(XV + bcat_ref[...] + zcat)

    xm = XV[:, 0:32]
    xs = xm[0:_R]
    for k in range(1, 15):
        xs = xs + xm[k * _R:(k + 1) * _R]
    xout_ref[0] = xs * jnp.float32(1.0 / 15.0)

    vm = XV[:, 32:96]
    part = jnp.sum(vm, axis=0, keepdims=True)
    acc = jnp.where(rb == 0, part, zout_ref[0] + part)
    zout_ref[0] = jnp.where(rb == _NB - 1,
                            acc * jnp.float32(1.0 / (15.0 * _NPTS)), acc)


def kernel(x, z, Wm2, bm2, Wm3, bm3, Wv2, bv2, Wv3, bv3,
           Wmx, bmx, Wvx, bvx, Wmz, bmz, Wvz, bvz):
    bs = x.shape[0]
    x3 = x.reshape(bs, _NPTS, _DF)
    xT = jnp.transpose(x3, (0, 2, 1))

    idxg, x3p = _run_topk_tc(x3, xT)
    neigh = _run_gather_sc(x3p, idxg.reshape(-1))
    neigh4 = neigh.reshape(bs, _NPTS, _NN, _PAD)

    row = lambda a: a.reshape(1, -1)
    z32 = jnp.zeros((32,), jnp.float32)
    cat = lambda parts: jnp.concatenate(parts).reshape(1, 128)
    A_row = cat([Wm2[:, 0], Wm3[:, 0], Wv2[:, 0], Wv3[:, 0]])
    B_row = cat([Wm2[:, 1], z32, Wv2[:, 1], z32])
    C_row = cat([z32, Wm3[:, 1], z32, Wv3[:, 1]])
    E_row = cat([z32, Wm3[:, 2], z32, Wv3[:, 2]])
    b4_row = cat([bm2, bm3, bv2, bv3])
    Wcat = jnp.zeros((128, 96), jnp.float32)
    Wcat = Wcat.at[0:64, 0:32].set(Wmx.T).at[64:128, 32:96].set(Wvx.T)
    bcat = jnp.concatenate([bmx, bvx]).reshape(1, 96)
    Wzc = jnp.concatenate([Wmz.T, Wvz.T], axis=1)
    bzc = jnp.concatenate([bmz, bvz]).reshape(1, 96)
    full = lambda shp: pl.BlockSpec(shp, lambda b, r: (0,) * len(shp))

    xout, zout = pl.pallas_call(
        _mlp_body,
        grid=(bs, _NB),
        in_specs=[
            pl.BlockSpec((1, _R, _NN, _PAD), lambda b, r: (b, r, 0, 0)),
            pl.BlockSpec((1, 1, 1), lambda b, r: (b, 0, 0)),
            full((1, 128)), full((1, 128)), full((1, 128)), full((1, 128)),
            full((1, 128)),
            full((128, 96)), full((1, 96)),
            full((1, 96)), full((1, 96)),
        ],
        out_specs=[
            pl.BlockSpec((1, _R, 32), lambda b, r: (b, r, 0)),
            pl.BlockSpec((1, 1, 64), lambda b, r: (b, 0, 0)),
        ],
        out_shape=[
            jax.ShapeDtypeStruct((bs, _NPTS, 32), jnp.float32),
            jax.ShapeDtypeStruct((bs, 1, 64), jnp.float32),
        ],
        compiler_params=pltpu.CompilerParams(
            dimension_semantics=("arbitrary", "arbitrary"),
        ),
    )(neigh4, z.reshape(bs, 1, 1),
      A_row, B_row, C_row, E_row, b4_row, Wcat, bcat, Wzc, bzc)

    return xout.reshape(bs, _NPTS * 32), zout.reshape(bs, 64)


# speedup vs baseline: 1.1367x; 1.0033x over previous
import jax
import jax.numpy as jnp
from jax import lax
from jax.experimental import pallas as pl
from jax.experimental.pallas import tpu as pltpu
from jax.experimental.pallas import tpu_sc as plsc

_BATCH = 8
_NPTS = 1024
_DF = 3
_NN = 16
_R = 512
_NB = _NPTS // _R
_RSC = 256
_NSC = _NPTS // _RSC
_PAD = 16
_BIG = 1e30


def _topk_body(x3_ref, xT_ref, idx_ref, x3p_ref):
    b = pl.program_id(0)
    q = x3_ref[0]
    X0 = xT_ref[0, 0:1, :]
    X1 = xT_ref[0, 1:2, :]
    X2 = xT_ref[0, 2:3, :]
    d0 = q[:, 0:1] - X0
    d1 = q[:, 1:2] - X1
    d2 = q[:, 2:3] - X2
    D = d0 * d0 + d1 * d1 + d2 * d2
    D = jnp.where(D == 0.0, _BIG, D)

    iota = lax.broadcasted_iota(jnp.int32, (_R, _NPTS), 1).astype(jnp.float32)
    cols = []
    for s in range(_NN):
        m = jnp.min(D, axis=1, keepdims=True)
        jsel = jnp.where(D == m, iota, jnp.float32(2.0 ** 30))
        jmin = jnp.min(jsel, axis=1, keepdims=True)
        cols.append(jmin)
        if s < _NN - 1:
            D = jnp.where(jsel == jmin, _BIG, D)
    idx = jnp.concatenate(cols, axis=1).astype(jnp.int32)
    idx_ref[0] = idx + b * _NPTS

    x3p_ref[...] = jnp.concatenate(
        [q, jnp.zeros((_R, _PAD - _DF), jnp.float32)], axis=1)


def _run_topk_tc(x3, xT):
    return pl.pallas_call(
        _topk_body,
        grid=(_BATCH, _NB),
        in_specs=[
            pl.BlockSpec((1, _R, _DF), lambda b, r: (b, r, 0)),
            pl.BlockSpec((1, _DF, _NPTS), lambda b, r: (b, 0, 0)),
        ],
        out_specs=[
            pl.BlockSpec((1, _R, _NN), lambda b, r: (b, r, 0)),
            pl.BlockSpec((_R, _PAD), lambda b, r: (b * _NB + r, 0)),
        ],
        out_shape=[
            jax.ShapeDtypeStruct((_BATCH, _NPTS, _NN), jnp.int32),
            jax.ShapeDtypeStruct((_BATCH * _NPTS, _PAD), jnp.float32),
        ],
        compiler_params=pltpu.CompilerParams(
            dimension_semantics=("arbitrary", "arbitrary"),
        ),
    )(x3, xT)


def _gather_sc(x3p_hbm, idx_hbm, out_hbm, ibuf_v, gbuf_v, semi, semg):
    wid = lax.axis_index("s") * 2 + lax.axis_index("c")
    b = wid // _NSC
    rb = wid % _NSC

    nseg = (_RSC * _NN) // 128
    base = (b * _NPTS + rb * _RSC) * _NN
    hs = []
    for r in range(nseg):
        hs.append(pltpu.async_copy(
            idx_hbm.at[pl.ds(base + r * 128, 128)],
            ibuf_v.at[r], semi))
    for h in hs:
        h.wait()
    hs = []
    for r in range(nseg):
        hs.append(pltpu.async_copy(
            x3p_hbm.at[ibuf_v.at[r]],
            gbuf_v.at[pl.ds(r * 128, 128)], semg))
    for h in hs:
        h.wait()
    pltpu.sync_copy(gbuf_v, out_hbm.at[b, pl.ds(rb * _RSC * _NN, _RSC * _NN)])


def _run_gather_sc(x3p, idxg):
    mesh = plsc.VectorSubcoreMesh(core_axis_name="c", subcore_axis_name="s")
    return pl.kernel(
        _gather_sc,
        mesh=mesh,
        out_type=jax.ShapeDtypeStruct((_BATCH, _NPTS * _NN, _PAD), jnp.float32),
        scratch_types=[
            pltpu.VMEM((32, 128), jnp.int32),
            pltpu.VMEM((_RSC * _NN, _PAD), jnp.float32),
            pltpu.SemaphoreType.DMA,
            pltpu.SemaphoreType.DMA,
        ],
        compiler_params=pltpu.CompilerParams(use_tc_tiling_on_sc=False),
    )(x3p, idxg)


def _mlp_body(nb_ref, z_ref, A_ref, B_ref, C_ref, E_ref, b4_ref,
              Wcat_ref, bcat_ref, Wzc_ref, bzc_ref,
              xout_ref, zout_ref):
    rb = pl.program_id(1)
    nb4 = nb_ref[0]

    def stack(cols):
        return jnp.concatenate(cols, axis=0)

    relu = lambda a: jnp.maximum(a, 0.0)
    third = jnp.float32(1.0 / 3.0)

    A = A_ref[...]
    B = B_ref[...]
    C = C_ref[...]
    E = E_ref[...]
    b4 = b4_ref[...]
    F = jnp.zeros((15 * _R, 128), jnp.float32)
    for f in range(_DF):
        cf = [nb4[:, s, f:f + 1] for s in range(_NN)]
        t1 = stack([cf[0]] * 15)
        t2 = stack(cf[1:16])
        u2 = stack([cf[1]] * 14 + [cf[2]])
        u3 = stack(cf[2:16] + [cf[3]])
        F = F + relu(t1 * A + t2 * B + u2 * C + u3 * E + b4)

    zsc = z_ref[0, 0, 0]
    zcat = zsc * Wzc_ref[...] + bzc_ref[...]

    XV = jnp.dot(F * third, Wcat_ref[...], preferred_element_type=jnp.float32)
    XV = relu(XV + bcat_ref[...] + zcat)

    xm = XV[:, 0:32]
    xs = xm[0:_R]
    for k in range(1, 15):
        xs = xs + xm[k * _R:(k + 1) * _R]
    xout_ref[0] = xs * jnp.float32(1.0 / 15.0)

    vm = XV[:, 32:96]
    part = jnp.sum(vm, axis=0, keepdims=True)
    acc = jnp.where(rb == 0, part, zout_ref[0] + part)
    zout_ref[0] = jnp.where(rb == _NB - 1,
                            acc * jnp.float32(1.0 / (15.0 * _NPTS)), acc)


def kernel(x, z, Wm2, bm2, Wm3, bm3, Wv2, bv2, Wv3, bv3,
           Wmx, bmx, Wvx, bvx, Wmz, bmz, Wvz, bvz):
    bs = x.shape[0]
    x3 = x.reshape(bs, _NPTS, _DF)
    xT = jnp.transpose(x3, (0, 2, 1))

    idxg, x3p = _run_topk_tc(x3, xT)
    neigh = _run_gather_sc(x3p, idxg.reshape(-1))
    neigh4 = neigh.reshape(bs, _NPTS, _NN, _PAD)

    row = lambda a: a.reshape(1, -1)
    z32 = jnp.zeros((32,), jnp.float32)
    cat = lambda parts: jnp.concatenate(parts).reshape(1, 128)
    A_row = cat([Wm2[:, 0], Wm3[:, 0], Wv2[:, 0], Wv3[:, 0]])
    B_row = cat([Wm2[:, 1], z32, Wv2[:, 1], z32])
    C_row = cat([z32, Wm3[:, 1], z32, Wv3[:, 1]])
    E_row = cat([z32, Wm3[:, 2], z32, Wv3[:, 2]])
    b4_row = cat([bm2, bm3, bv2, bv3])
    Wcat = jnp.zeros((128, 96), jnp.float32)
    Wcat = Wcat.at[0:64, 0:32].set(Wmx.T).at[64:128, 32:96].set(Wvx.T)
    bcat = jnp.concatenate([bmx, bvx]).reshape(1, 96)
    Wzc = jnp.concatenate([Wmz.T, Wvz.T], axis=1)
    bzc = jnp.concatenate([bmz, bvz]).reshape(1, 96)
    full = lambda shp: pl.BlockSpec(shp, lambda b, r: (0,) * len(shp))

    xout, zout = pl.pallas_call(
        _mlp_body,
        grid=(bs, _NB),
        in_specs=[
            pl.BlockSpec((1, _R, _NN, _PAD), lambda b, r: (b, r, 0, 0)),
            pl.BlockSpec((1, 1, 1), lambda b, r: (b, 0, 0)),
            full((1, 128)), full((1, 128)), full((1, 128)), full((1, 128)),
            full((1, 128)),
            full((128, 96)), full((1, 96)),
            full((1, 96)), full((1, 96)),
        ],
        out_specs=[
            pl.BlockSpec((1, _R, 32), lambda b, r: (b, r, 0)),
            pl.BlockSpec((1, 1, 64), lambda b, r: (b, 0, 0)),
        ],
        out_shape=[
            jax.ShapeDtypeStruct((bs, _NPTS, 32), jnp.float32),
            jax.ShapeDtypeStruct((bs, 1, 64), jnp.float32),
        ],
        compiler_params=pltpu.CompilerParams(
            dimension_semantics=("arbitrary", "arbitrary"),
        ),
    )(neigh4, z.reshape(bs, 1, 1),
      A_row, B_row, C_row, E_row, b4_row, Wcat, bcat, Wzc, bzc)

    return xout.reshape(bs, _NPTS * 32), zout.reshape(bs, 64)
